# baseline (speedup 1.0000x reference)
import math

import jax
import jax.numpy as jnp
from jax import lax
from jax.experimental import pallas as pl
from jax.experimental.pallas import tpu as pltpu
from jax.experimental.pallas import tpu_sc as plsc

_D = 64
_L = 16
_NC, _NS = 2, 16
_NW = _NC * _NS
_BPW = 128
_NBUF = 4


def kernel(x, table):
    nb, h = x.shape
    scale = jnp.float32(math.sqrt(_D))
    nft = _D // 8
    ntile = nb // _BPW

    def body(x_hbm, tab_hbm, out_hbm, idx_v, idx_t, raw_v, tp_v,
             g0, g1, g2, g3, o0, o1, o2, o3):
        gs = [g0, g1, g2, g3]
        os_ = [o0, o1, o2, o3]
        wid = lax.axis_index("s") * _NC + lax.axis_index("c")
        b0 = wid * _BPW
        pltpu.sync_copy(x_hbm.at[pl.ds(b0, _BPW)], idx_v)

        iota = lax.iota(jnp.int32, _L)
        rowsel = [iota + jnp.int32(16 * bj) for bj in range(_BPW // _L)]

        def tbody(s, c):
            for bj in range(_BPW // _L):
                v = plsc.load_gather(idx_v, [rowsel[bj], jnp.full((_L,), s, jnp.int32)])
                idx_t[s, pl.ds(16 * bj, _L)] = v
            return c

        lax.fori_loop(0, h, tbody, 0)

        def issue_gather(ci, bb):
            pltpu.async_copy(tab_hbm.at[idx_t.at[ci]], raw_v.at[bb], gs[bb])

        def drain_gather(bb):
            pltpu.make_async_copy(
                tab_hbm.at[pl.ds(0, _BPW)], raw_v.at[bb], gs[bb]
            ).wait()

        def transform_buf(bb):
            def ftbody(ft, c):
                for fi in range(8):
                    col = jnp.full((_L,), 8 * ft + fi, jnp.int32)
                    for bj in range(_BPW // _L):
                        v = plsc.load_gather(raw_v.at[bb], [rowsel[bj], col])
                        tp_v[bb, ft, fi, pl.ds(16 * bj, _L)] = v * scale
                return c

            lax.fori_loop(0, nft, ftbody, 0)

        def issue_out(ci, bb):
            pltpu.async_copy(
                tp_v.at[bb],
                out_hbm.at[ci, :, wid],
                os_[bb],
            )

        def drain_out(bb):
            pltpu.make_async_copy(
                tp_v.at[bb],
                out_hbm.at[0, :, 0],
                os_[bb],
            ).wait()

        for bb in range(_NBUF):
            issue_gather(bb, bb)

        def group(gi, c):
            i0 = gi * _NBUF
            for bb in range(_NBUF):
                ci = i0 + bb
                drain_gather(bb)
                transform_buf(bb)
                issue_out(ci, bb)
                drain_out(bb)
                issue_gather(ci + _NBUF, bb)
            return c

        lax.fori_loop(0, h // _NBUF - 1, group, 0)

        i0 = h - _NBUF
        for bb in range(_NBUF):
            drain_gather(bb)
            transform_buf(bb)
            issue_out(i0 + bb, bb)
        for bb in range(_NBUF):
            drain_out(bb)

    out5 = pl.kernel(
        body,
        out_type=jax.ShapeDtypeStruct((h, nft, ntile, 8, _BPW), jnp.float32),
        mesh=plsc.VectorSubcoreMesh(core_axis_name="c", subcore_axis_name="s"),
        compiler_params=pltpu.CompilerParams(
            use_tc_tiling_on_sc=False, needs_layout_passes=False
        ),
        scratch_types=[
            pltpu.VMEM((_BPW, h), jnp.int32),
            pltpu.VMEM((h, _BPW), jnp.int32),
            pltpu.VMEM((_NBUF, _BPW, _D), jnp.float32),
            pltpu.VMEM((_NBUF, nft, 8, _BPW), jnp.float32),
        ] + [pltpu.SemaphoreType.DMA] * (2 * _NBUF),
    )(x, table)

    return lax.reshape(out5, (nb, h, _D), dimensions=(2, 4, 0, 1, 3))


# speedup vs baseline: 1.7542x; 1.7542x over previous
import math

import jax
import jax.numpy as jnp
from jax import lax
from jax.experimental import pallas as pl
from jax.experimental.pallas import tpu as pltpu
from jax.experimental.pallas import tpu_sc as plsc

_D = 64
_L = 16
_NC, _NS = 2, 16
_NW = _NC * _NS
_BPW = 128
_NBUF = 4
_PITCH = _BPW + 1


def kernel(x, table):
    nb, h = x.shape
    scale = jnp.float32(math.sqrt(_D))
    nft = _D // 8
    ntile = nb // _BPW

    def body(x_hbm, tab_hbm, out_hbm, idx_v, idx_t, raw_v, tp_v,
             g0, g1, g2, g3, o0, o1, o2, o3):
        gs = [g0, g1, g2, g3]
        os_ = [o0, o1, o2, o3]
        wid = lax.axis_index("s") * _NC + lax.axis_index("c")
        b0 = wid * _BPW
        pltpu.sync_copy(x_hbm.at[pl.ds(b0, _BPW)], idx_v)

        iota = lax.iota(jnp.int32, _L)
        rowsel = [iota + jnp.int32(16 * bj) for bj in range(_BPW // _L)]
        ftsel = [(jnp.int32(16 * c) + iota) // jnp.int32(8) for c in range(_D // _L)]
        fisel = [(jnp.int32(16 * c) + iota) % jnp.int32(8) for c in range(_D // _L)]

        def tbody(s, c):
            for bj in range(_BPW // _L):
                v = plsc.load_gather(idx_v, [rowsel[bj], jnp.full((_L,), s, jnp.int32)])
                idx_t[s, pl.ds(16 * bj, _L)] = v
            return c

        lax.fori_loop(0, h, tbody, 0)

        def issue_gather(ci, bb):
            pltpu.async_copy(tab_hbm.at[idx_t.at[ci]], raw_v.at[bb], gs[bb])

        def drain_gather(bb):
            pltpu.make_async_copy(
                tab_hbm.at[pl.ds(0, _BPW)], raw_v.at[bb], gs[bb]
            ).wait()

        def transform_buf(bb):
            def rbody(r, c):
                col = jnp.full((_L,), r, jnp.int32)
                for cc in range(_D // _L):
                    v = raw_v[bb, r, pl.ds(16 * cc, _L)] * scale
                    plsc.store_scatter(tp_v.at[bb], [ftsel[cc], fisel[cc], col], v)
                return c

            lax.fori_loop(0, _BPW, rbody, 0)

        def issue_out(ci, bb):
            pltpu.async_copy(
                tp_v.at[bb, :, :, pl.ds(0, _BPW)],
                out_hbm.at[ci, :, wid],
                os_[bb],
            )

        def drain_out(bb):
            pltpu.make_async_copy(
                tp_v.at[bb, :, :, pl.ds(0, _BPW)],
                out_hbm.at[0, :, 0],
                os_[bb],
            ).wait()

        for bb in range(_NBUF):
            issue_gather(bb, bb)

        def group(gi, c):
            i0 = gi * _NBUF
            for bb in range(_NBUF):
                ci = i0 + bb
                drain_gather(bb)
                transform_buf(bb)
                issue_out(ci, bb)
                drain_out(bb)
                issue_gather(ci + _NBUF, bb)
            return c

        lax.fori_loop(0, h // _NBUF - 1, group, 0)

        i0 = h - _NBUF
        for bb in range(_NBUF):
            drain_gather(bb)
            transform_buf(bb)
            issue_out(i0 + bb, bb)
        for bb in range(_NBUF):
            drain_out(bb)

    out5 = pl.kernel(
        body,
        out_type=jax.ShapeDtypeStruct((h, nft, ntile, 8, _BPW), jnp.float32),
        mesh=plsc.VectorSubcoreMesh(core_axis_name="c", subcore_axis_name="s"),
        compiler_params=pltpu.CompilerParams(
            use_tc_tiling_on_sc=False, needs_layout_passes=False
        ),
        scratch_types=[
            pltpu.VMEM((_BPW, h), jnp.int32),
            pltpu.VMEM((h, _BPW), jnp.int32),
            pltpu.VMEM((_NBUF, _BPW, _D), jnp.float32),
            pltpu.VMEM((_NBUF, nft, 8, _PITCH), jnp.float32),
        ] + [pltpu.SemaphoreType.DMA] * (2 * _NBUF),
    )(x, table)

    return lax.reshape(out5, (nb, h, _D), dimensions=(2, 4, 0, 1, 3))


# speedup vs baseline: 2.7268x; 1.5545x over previous
import math

import jax
import jax.numpy as jnp
from jax import lax
from jax.experimental import pallas as pl
from jax.experimental.pallas import tpu as pltpu
from jax.experimental.pallas import tpu_sc as plsc

_D = 64
_L = 16
_NC, _NS = 2, 16
_NW = _NC * _NS
_BPW = 128
_NBUF = 4
_PITCH = _BPW + 1


def kernel(x, table):
    nb, h = x.shape
    scale = jnp.float32(math.sqrt(_D))
    nft = _D // 8
    ntile = nb // _BPW

    def body(x_hbm, tab_hbm, out_hbm, idx_v, idx_t, raw_v, tp_v,
             g0, g1, g2, g3, o0, o1, o2, o3):
        gs = [g0, g1, g2, g3]
        os_ = [o0, o1, o2, o3]
        wid = lax.axis_index("s") * _NC + lax.axis_index("c")
        b0 = wid * _BPW
        pltpu.sync_copy(x_hbm.at[pl.ds(b0, _BPW)], idx_v)

        iota = lax.iota(jnp.int32, _L)
        rowsel = [iota + jnp.int32(16 * bj) for bj in range(_BPW // _L)]
        ftsel = [(jnp.int32(16 * c) + iota) // jnp.int32(8) for c in range(_D // _L)]
        fisel = [(jnp.int32(16 * c) + iota) % jnp.int32(8) for c in range(_D // _L)]

        def tbody(s, c):
            for bj in range(_BPW // _L):
                v = plsc.load_gather(idx_v, [rowsel[bj], jnp.full((_L,), s, jnp.int32)])
                idx_t[s, pl.ds(16 * bj, _L)] = v
            return c

        lax.fori_loop(0, h, tbody, 0)

        def issue_gather(ci, bb):
            pltpu.async_copy(tab_hbm.at[idx_t.at[ci]], raw_v.at[bb], gs[bb])

        def drain_gather(bb):
            pltpu.make_async_copy(
                tab_hbm.at[pl.ds(0, _BPW)], raw_v.at[bb], gs[bb]
            ).wait()

        def transform_buf(bb):
            @plsc.parallel_loop(0, _BPW, unroll=8)
            def rbody(r):
                col = jnp.full((_L,), r, jnp.int32)
                for cc in range(_D // _L):
                    v = raw_v[bb, r, pl.ds(16 * cc, _L)] * scale
                    plsc.store_scatter(tp_v.at[bb], [ftsel[cc], fisel[cc], col], v)

        def issue_out(ci, bb):
            pltpu.async_copy(
                tp_v.at[bb, :, :, pl.ds(0, _BPW)],
                out_hbm.at[ci, :, wid],
                os_[bb],
            )

        def drain_out(bb):
            pltpu.make_async_copy(
                tp_v.at[bb, :, :, pl.ds(0, _BPW)],
                out_hbm.at[0, :, 0],
                os_[bb],
            ).wait()

        for bb in range(_NBUF):
            issue_gather(bb, bb)

        def group(gi, c):
            i0 = gi * _NBUF
            for bb in range(_NBUF):
                ci = i0 + bb
                drain_gather(bb)

                @pl.when(gi > 0)
                def _():
                    drain_out(bb)

                transform_buf(bb)
                issue_out(ci, bb)
                issue_gather(ci + _NBUF, bb)
            return c

        lax.fori_loop(0, h // _NBUF - 1, group, 0)

        i0 = h - _NBUF
        for bb in range(_NBUF):
            drain_gather(bb)
            drain_out(bb)
            transform_buf(bb)
            issue_out(i0 + bb, bb)
        for bb in range(_NBUF):
            drain_out(bb)

    out5 = pl.kernel(
        body,
        out_type=jax.ShapeDtypeStruct((h, nft, ntile, 8, _BPW), jnp.float32),
        mesh=plsc.VectorSubcoreMesh(core_axis_name="c", subcore_axis_name="s"),
        compiler_params=pltpu.CompilerParams(
            use_tc_tiling_on_sc=False, needs_layout_passes=False
        ),
        scratch_types=[
            pltpu.VMEM((_BPW, h), jnp.int32),
            pltpu.VMEM((h, _BPW), jnp.int32),
            pltpu.VMEM((_NBUF, _BPW, _D), jnp.float32),
            pltpu.VMEM((_NBUF, nft, 8, _PITCH), jnp.float32),
        ] + [pltpu.SemaphoreType.DMA] * (2 * _NBUF),
    )(x, table)

    return lax.reshape(out5, (nb, h, _D), dimensions=(2, 4, 0, 1, 3))


# speedup vs baseline: 2.7324x; 1.0021x over previous
import math

import jax
import jax.numpy as jnp
from jax import lax
from jax.experimental import pallas as pl
from jax.experimental.pallas import tpu as pltpu
from jax.experimental.pallas import tpu_sc as plsc

_D = 64
_L = 16
_NC, _NS = 2, 16
_NW = _NC * _NS
_BPW = 128
_NBUF = 4
_PITCH = _BPW + 1


def kernel(x, table):
    nb, h = x.shape
    scale = jnp.float32(math.sqrt(_D))
    nft = _D // 8
    ntile = nb // _BPW

    def body(x_hbm, tab_hbm, out_hbm, idx_v, idx_t, raw_v, tp_v,
             g0, g1, g2, g3, o0, o1, o2, o3):
        gs = [g0, g1, g2, g3]
        os_ = [o0, o1, o2, o3]
        wid = lax.axis_index("s") * _NC + lax.axis_index("c")
        b0 = wid * _BPW
        pltpu.sync_copy(x_hbm.at[pl.ds(b0, _BPW)], idx_v)

        iota = lax.iota(jnp.int32, _L)
        rowsel = [iota + jnp.int32(16 * bj) for bj in range(_BPW // _L)]
        ftsel = [(jnp.int32(16 * c) + iota) // jnp.int32(8) for c in range(_D // _L)]
        fisel = [(jnp.int32(16 * c) + iota) % jnp.int32(8) for c in range(_D // _L)]

        def tbody(s, c):
            for bj in range(_BPW // _L):
                v = plsc.load_gather(idx_v, [rowsel[bj], jnp.full((_L,), s, jnp.int32)])
                idx_t[s, pl.ds(16 * bj, _L)] = v
            return c

        lax.fori_loop(0, h, tbody, 0)

        def issue_gather(ci, bb):
            pltpu.async_copy(tab_hbm.at[idx_t.at[ci]], raw_v.at[bb], gs[bb])

        def drain_gather(bb):
            pltpu.make_async_copy(
                tab_hbm.at[pl.ds(0, _BPW)], raw_v.at[bb], gs[bb]
            ).wait()

        def transform_buf(bb):
            @plsc.parallel_loop(0, _BPW, unroll=16)
            def rbody(r):
                col = jnp.full((_L,), r, jnp.int32)
                for cc in range(_D // _L):
                    v = raw_v[bb, r, pl.ds(16 * cc, _L)] * scale
                    plsc.store_scatter(tp_v.at[bb], [ftsel[cc], fisel[cc], col], v)

        def issue_out(ci, bb):
            pltpu.async_copy(
                tp_v.at[bb, :, :, pl.ds(0, _BPW)],
                out_hbm.at[ci, :, wid],
                os_[bb],
            )

        def drain_out(bb):
            pltpu.make_async_copy(
                tp_v.at[bb, :, :, pl.ds(0, _BPW)],
                out_hbm.at[0, :, 0],
                os_[bb],
            ).wait()

        for bb in range(_NBUF):
            issue_gather(bb, bb)

        def group(gi, c):
            i0 = gi * _NBUF
            for bb in range(_NBUF):
                ci = i0 + bb
                drain_gather(bb)

                @pl.when(gi > 0)
                def _():
                    drain_out(bb)

                transform_buf(bb)
                issue_out(ci, bb)
                issue_gather(ci + _NBUF, bb)
            return c

        lax.fori_loop(0, h // _NBUF - 1, group, 0)

        i0 = h - _NBUF
        for bb in range(_NBUF):
            drain_gather(bb)
            drain_out(bb)
            transform_buf(bb)
            issue_out(i0 + bb, bb)
        for bb in range(_NBUF):
            drain_out(bb)

    out5 = pl.kernel(
        body,
        out_type=jax.ShapeDtypeStruct((h, nft, ntile, 8, _BPW), jnp.float32),
        mesh=plsc.VectorSubcoreMesh(core_axis_name="c", subcore_axis_name="s"),
        compiler_params=pltpu.CompilerParams(
            use_tc_tiling_on_sc=False, needs_layout_passes=False
        ),
        scratch_types=[
            pltpu.VMEM((_BPW, h), jnp.int32),
            pltpu.VMEM((h, _BPW), jnp.int32),
            pltpu.VMEM((_NBUF, _BPW, _D), jnp.float32),
            pltpu.VMEM((_NBUF, nft, 8, _PITCH), jnp.float32),
        ] + [pltpu.SemaphoreType.DMA] * (2 * _NBUF),
    )(x, table)

    return lax.reshape(out5, (nb, h, _D), dimensions=(2, 4, 0, 1, 3))


# speedup vs baseline: 2.7611x; 1.0105x over previous
import math

import jax
import jax.numpy as jnp
from jax import lax
from jax.experimental import pallas as pl
from jax.experimental.pallas import tpu as pltpu
from jax.experimental.pallas import tpu_sc as plsc

_D = 64
_L = 16
_NC, _NS = 2, 16
_NW = _NC * _NS
_BPW = 128
_NBUF = 4
_PITCH = _BPW + 1


def kernel(x, table):
    nb, h = x.shape
    scale = jnp.float32(math.sqrt(_D))
    nft = _D // 8
    ntile = nb // _BPW

    def body(x_hbm, tab_hbm, out_hbm, idx_v, idx_t, raw_v, tp_v,
             g0, g1, g2, g3, o0, o1, o2, o3):
        gs = [g0, g1, g2, g3]
        os_ = [o0, o1, o2, o3]
        wid = lax.axis_index("s") * _NC + lax.axis_index("c")
        b0 = wid * _BPW
        pltpu.sync_copy(x_hbm.at[pl.ds(b0, _BPW)], idx_v)

        iota = lax.iota(jnp.int32, _L)
        rowsel = [iota + jnp.int32(16 * bj) for bj in range(_BPW // _L)]
        ftsel = [(jnp.int32(16 * c) + iota) // jnp.int32(8) for c in range(_D // _L)]
        fisel = [(jnp.int32(16 * c) + iota) % jnp.int32(8) for c in range(_D // _L)]

        def ttrans(s):
            for bj in range(_BPW // _L):
                v = plsc.load_gather(idx_v, [rowsel[bj], jnp.full((_L,), s, jnp.int32)])
                idx_t[s, pl.ds(16 * bj, _L)] = v

        def issue_gather(ci, bb):
            pltpu.async_copy(tab_hbm.at[idx_t.at[ci]], raw_v.at[bb], gs[bb])

        def drain_gather(bb):
            pltpu.make_async_copy(
                tab_hbm.at[pl.ds(0, _BPW)], raw_v.at[bb], gs[bb]
            ).wait()

        def transform_buf(bb):
            @plsc.parallel_loop(0, _BPW, unroll=16)
            def rbody(r):
                col = jnp.full((_L,), r, jnp.int32)
                for cc in range(_D // _L):
                    v = raw_v[bb, r, pl.ds(16 * cc, _L)] * scale
                    plsc.store_scatter(tp_v.at[bb], [ftsel[cc], fisel[cc], col], v)

        def issue_out(ci, bb):
            pltpu.async_copy(
                tp_v.at[bb, :, :, pl.ds(0, _BPW)],
                out_hbm.at[ci, :, wid],
                os_[bb],
            )

        def drain_out(bb):
            pltpu.make_async_copy(
                tp_v.at[bb, :, :, pl.ds(0, _BPW)],
                out_hbm.at[0, :, 0],
                os_[bb],
            ).wait()

        for bb in range(_NBUF):
            ttrans(jnp.int32(bb))
            issue_gather(bb, bb)

        @plsc.parallel_loop(_NBUF, h, unroll=4)
        def _(s):
            ttrans(s)

        def group(gi, c):
            i0 = gi * _NBUF
            for bb in range(_NBUF):
                ci = i0 + bb
                drain_gather(bb)

                @pl.when(gi > 0)
                def _():
                    drain_out(bb)

                transform_buf(bb)
                issue_out(ci, bb)
                issue_gather(ci + _NBUF, bb)
            return c

        lax.fori_loop(0, h // _NBUF - 1, group, 0)

        i0 = h - _NBUF
        for bb in range(_NBUF):
            drain_gather(bb)
            drain_out(bb)
            transform_buf(bb)
            issue_out(i0 + bb, bb)
        for bb in range(_NBUF):
            drain_out(bb)

    out5 = pl.kernel(
        body,
        out_type=jax.ShapeDtypeStruct((h, nft, ntile, 8, _BPW), jnp.float32),
        mesh=plsc.VectorSubcoreMesh(core_axis_name="c", subcore_axis_name="s"),
        compiler_params=pltpu.CompilerParams(
            use_tc_tiling_on_sc=False, needs_layout_passes=False
        ),
        scratch_types=[
            pltpu.VMEM((_BPW, h), jnp.int32),
            pltpu.VMEM((h, _BPW), jnp.int32),
            pltpu.VMEM((_NBUF, _BPW, _D), jnp.float32),
            pltpu.VMEM((_NBUF, nft, 8, _PITCH), jnp.float32),
        ] + [pltpu.SemaphoreType.DMA] * (2 * _NBUF),
    )(x, table)

    return lax.reshape(out5, (nb, h, _D), dimensions=(2, 4, 0, 1, 3))


# speedup vs baseline: 2.7678x; 1.0024x over previous
import math

import jax
import jax.numpy as jnp
from jax import lax
from jax.experimental import pallas as pl
from jax.experimental.pallas import tpu as pltpu
from jax.experimental.pallas import tpu_sc as plsc

_D = 64
_L = 16
_NC, _NS = 2, 16
_NW = _NC * _NS
_BPW = 128
_NBUF = 4
_PITCH = _BPW + 1


def kernel(x, table):
    nb, h = x.shape
    scale = jnp.float32(math.sqrt(_D))
    nft = _D // 8
    ntile = nb // _BPW

    def body(x_hbm, tab_hbm, out_hbm, idx_t, raw_v, tp_v,
             g0, g1, g2, g3, o0, o1, o2, o3):
        gs = [g0, g1, g2, g3]
        os_ = [o0, o1, o2, o3]
        wid = lax.axis_index("s") * _NC + lax.axis_index("c")
        pltpu.sync_copy(x_hbm.at[:, wid], idx_t)

        iota = lax.iota(jnp.int32, _L)
        ftsel = [(jnp.int32(16 * c) + iota) // jnp.int32(8) for c in range(_D // _L)]
        fisel = [(jnp.int32(16 * c) + iota) % jnp.int32(8) for c in range(_D // _L)]

        def issue_gather(ci, bb):
            pltpu.async_copy(
                tab_hbm.at[idx_t.at[ci // 8, ci % 8]], raw_v.at[bb], gs[bb]
            )

        def drain_gather(bb):
            pltpu.make_async_copy(
                tab_hbm.at[pl.ds(0, _BPW)], raw_v.at[bb], gs[bb]
            ).wait()

        def transform_buf(bb):
            @plsc.parallel_loop(0, _BPW, unroll=16)
            def rbody(r):
                col = jnp.full((_L,), r, jnp.int32)
                for cc in range(_D // _L):
                    v = raw_v[bb, r, pl.ds(16 * cc, _L)] * scale
                    plsc.store_scatter(tp_v.at[bb], [ftsel[cc], fisel[cc], col], v)

        def issue_out(ci, bb):
            pltpu.async_copy(
                tp_v.at[bb, :, :, pl.ds(0, _BPW)],
                out_hbm.at[ci, :, wid],
                os_[bb],
            )

        def drain_out(bb):
            pltpu.make_async_copy(
                tp_v.at[bb, :, :, pl.ds(0, _BPW)],
                out_hbm.at[0, :, 0],
                os_[bb],
            ).wait()

        for bb in range(_NBUF):
            issue_gather(bb, bb)

        def group(gi, c):
            i0 = gi * _NBUF
            for bb in range(_NBUF):
                ci = i0 + bb
                drain_gather(bb)

                @pl.when(gi > 0)
                def _():
                    drain_out(bb)

                transform_buf(bb)
                issue_out(ci, bb)
                issue_gather(ci + _NBUF, bb)
            return c

        lax.fori_loop(0, h // _NBUF - 1, group, 0)

        i0 = h - _NBUF
        for bb in range(_NBUF):
            drain_gather(bb)
            drain_out(bb)
            transform_buf(bb)
            issue_out(i0 + bb, bb)
        for bb in range(_NBUF):
            drain_out(bb)

    xv = (
        x.transpose(1, 0)
        .reshape(h // 8, 8, ntile, _BPW)
        .transpose(0, 2, 1, 3)
    )

    out5 = pl.kernel(
        body,
        out_type=jax.ShapeDtypeStruct((h, nft, ntile, 8, _BPW), jnp.float32),
        mesh=plsc.VectorSubcoreMesh(core_axis_name="c", subcore_axis_name="s"),
        compiler_params=pltpu.CompilerParams(
            use_tc_tiling_on_sc=False, needs_layout_passes=False
        ),
        scratch_types=[
            pltpu.VMEM((h // 8, 8, _BPW), jnp.int32),
            pltpu.VMEM((_NBUF, _BPW, _D), jnp.float32),
            pltpu.VMEM((_NBUF, nft, 8, _PITCH), jnp.float32),
        ] + [pltpu.SemaphoreType.DMA] * (2 * _NBUF),
    )(xv, table)

    return lax.reshape(out5, (nb, h, _D), dimensions=(2, 4, 0, 1, 3))
